# Initial kernel scaffold; baseline (speedup 1.0000x reference)
import jax
import jax.numpy as jnp
from jax.experimental import pallas as pl


def kernel(x, edge_index, W, b):
    raise NotImplementedError("write your pallas kernel here")


# speedup vs baseline: 5.0815x; 5.0815x over previous
import functools

import jax
import jax.numpy as jnp
from jax import lax
from jax.experimental import pallas as pl
from jax.experimental.pallas import tpu as pltpu, tpu_sc as plsc

MIN_NORM = 1e-15
PROJ_EPS = 4e-3

_N = 10000
_D = 128
_E = 320000

_NW = 32
_CHUNK = 128
_NCHUNK = -(-_E // (_NW * _CHUNK))
_EPT = _NCHUNK * _CHUNK
_EPAD = _NW * _EPT
_RPT = 8 * (-(-(_N + 1) // (16 * 8)))
_NR = _RPT * 16

_BLK = 1000


def _norm(x):
    return jnp.maximum(jnp.sqrt(jnp.sum(x * x, axis=-1, keepdims=True)), MIN_NORM)


def _artanh(x):
    x = jnp.clip(x, -1.0 + 1e-7, 1.0 - 1e-7)
    return 0.5 * jnp.log((1.0 + x) / (1.0 - x))


def _proj(x):
    n = _norm(x)
    maxnorm = 1.0 - PROJ_EPS
    return jnp.where(n > maxnorm, x / n * maxnorm, x)


def _expmap0(u):
    n = _norm(u)
    return jnp.tanh(n) * u / n


def _logmap0(p):
    n = _norm(p)
    return p / n * _artanh(n)


def _pre_body(x_ref, w_ref, b_ref, o_ref):
    x = x_ref[...]
    w = w_ref[...]
    b = b_ref[...]
    x_hyp = _proj(_expmap0(x))
    x_norm = _norm(x_hyp)
    mx = lax.dot_general(x_hyp, w, (((1,), (1,)), ((), ())),
                         preferred_element_type=jnp.float32)
    mx_norm = _norm(mx)
    res = jnp.tanh(mx_norm / x_norm * _artanh(x_norm)) * mx / mx_norm
    cond = jnp.all(mx == 0, axis=-1, keepdims=True)
    mv = _proj(jnp.where(cond, jnp.zeros_like(res), res))
    bias_hyp = _proj(_expmap0(b))
    x2 = jnp.sum(mv * mv, axis=-1, keepdims=True)
    y2 = jnp.sum(bias_hyp * bias_hyp, axis=-1, keepdims=True)
    xy = jnp.sum(mv * bias_hyp, axis=-1, keepdims=True)
    num = (1.0 + 2.0 * xy + y2) * mv + (1.0 - x2) * bias_hyp
    den = 1.0 + 2.0 * xy + x2 * y2
    h = _proj(num / jnp.maximum(den, MIN_NORM))
    o_ref[...] = _logmap0(h)


def _post_body(p_ref, o_ref):
    agg = p_ref[0] + p_ref[1]
    h = _proj(_expmap0(agg))
    h_tan = jnp.maximum(_logmap0(h), 0.0)
    h = _proj(_expmap0(h_tan))
    o_ref[...] = _logmap0(h)


def _agg_body(src_hbm, dst_hbm, htan_hbm, zeros_hbm, out_hbm,
              src_v, dst_v, rows_v, acc_sh, sem):
    cid = lax.axis_index("c")
    sid = lax.axis_index("s")
    wid = sid * 2 + cid
    r0 = sid * _RPT
    pltpu.sync_copy(zeros_hbm.at[pl.ds(r0, _RPT)], acc_sh.at[pl.ds(r0, _RPT)])
    pltpu.sync_copy(src_hbm.at[wid], src_v)
    pltpu.sync_copy(dst_hbm.at[wid], dst_v)
    plsc.subcore_barrier()

    def chunk(j, carry):
        pltpu.async_copy(htan_hbm.at[src_v.at[j]], rows_v, sem).wait()
        pltpu.sync_copy(rows_v, acc_sh.at[dst_v.at[j]], add=True)
        return carry

    lax.fori_loop(0, _NCHUNK, chunk, 0)
    plsc.subcore_barrier()
    pltpu.sync_copy(acc_sh.at[pl.ds(r0, _RPT)],
                    out_hbm.at[cid, pl.ds(r0, _RPT)])


def _make_agg_call():
    return functools.partial(
        pl.kernel,
        out_type=jax.ShapeDtypeStruct((2, _NR, _D), jnp.float32),
        mesh=plsc.VectorSubcoreMesh(core_axis_name="c", subcore_axis_name="s"),
        scratch_types=[
            pltpu.VMEM((_NCHUNK, _CHUNK), jnp.int32),
            pltpu.VMEM((_NCHUNK, _CHUNK), jnp.int32),
            pltpu.VMEM((_CHUNK, _D), jnp.float32),
            pltpu.VMEM_SHARED((_NR, _D), jnp.float32),
            pltpu.SemaphoreType.DMA,
        ],
    )(_agg_body)


def kernel(x, edge_index, W, b):
    n_blocks = _N // _BLK
    h_tan = pl.pallas_call(
        _pre_body,
        grid=(n_blocks,),
        in_specs=[
            pl.BlockSpec((_BLK, _D), lambda i: (i, 0)),
            pl.BlockSpec((_D, _D), lambda i: (0, 0)),
            pl.BlockSpec((1, _D), lambda i: (0, 0)),
        ],
        out_specs=pl.BlockSpec((_BLK, _D), lambda i: (i, 0)),
        out_shape=jax.ShapeDtypeStruct((_N, _D), jnp.float32),
    )(x, W, b.reshape(1, _D))

    pad = _EPAD - _E
    src = jnp.concatenate([edge_index[0], jnp.zeros((pad,), jnp.int32)])
    dst = jnp.concatenate([edge_index[1], jnp.full((pad,), _N, jnp.int32)])
    src_r = src.reshape(_NW, _NCHUNK, _CHUNK)
    dst_r = dst.reshape(_NW, _NCHUNK, _CHUNK)
    zeros = jnp.zeros((_NR, _D), jnp.float32)

    partials = _make_agg_call()(src_r, dst_r, h_tan, zeros)

    out = pl.pallas_call(
        _post_body,
        grid=(n_blocks,),
        in_specs=[pl.BlockSpec((2, _BLK, _D), lambda i: (0, i, 0))],
        out_specs=pl.BlockSpec((_BLK, _D), lambda i: (i, 0)),
        out_shape=jax.ShapeDtypeStruct((_N, _D), jnp.float32),
    )(partials)
    return out
